# baseline (speedup 1.0000x reference)
import functools

import jax
import jax.numpy as jnp
from jax import lax
from jax.experimental import pallas as pl
from jax.experimental.pallas import tpu as pltpu
from jax.experimental.pallas import tpu_sc as plsc

B = 16384
N_SPARSE = 26
DENSE_DIMS = 13
NCOL = N_SPARSE + DENSE_DIMS
VOCAB = 100000
NC = 2
NSUB = 16
NW = NC * NSUB
RPW = B // NW
LANES = 16
CHUNKS = RPW // LANES
HALF = 13
HGROUPS = (0, 5, 9, 13)
HNG = len(HGROUPS) - 1


def _tc_prep_body(rawT_ref, w_ref, idx_ref, densep_ref):
    for f in range(N_SPARSE):
        idx_ref[pl.ds(f * B, B)] = rawT_ref[f, :].astype(jnp.int32) + f * VOCAB
    acc = rawT_ref[N_SPARSE, :] * w_ref[0, 0]
    for d in range(1, DENSE_DIMS):
        acc = acc + rawT_ref[N_SPARSE + d, :] * w_ref[0, d]
    densep_ref[...] = acc


@functools.cache
def _build_tc_prep():
    return pl.pallas_call(
        _tc_prep_body,
        out_shape=(jax.ShapeDtypeStruct((N_SPARSE * B,), jnp.int32),
                   jax.ShapeDtypeStruct((B,), jnp.float32)),
    )


def _sc_gather_half_body(f0, idx_hbm, init_hbm, table_hbm, out_hbm,
                         idxb, vals, outv, sem0, sem1):
    wid = lax.axis_index("s") * NC + lax.axis_index("c")
    base = wid * RPW
    sems = (sem0, sem1)
    idescs = [
        pltpu.async_copy(idx_hbm.at[pl.ds((f0 + f) * B + base, RPW)],
                         idxb.at[pl.ds(f * RPW, RPW)], sem0)
        for f in range(HALF)
    ]
    ind = pltpu.async_copy(init_hbm.at[pl.ds(base, RPW)], outv, sem1)
    for d in idescs:
        d.wait()
    ind.wait()

    def fire(g):
        lo, hi = HGROUPS[g], HGROUPS[g + 1]
        n = (hi - lo) * RPW
        return pltpu.async_copy(table_hbm.at[idxb.at[pl.ds(lo * RPW, n)]],
                                vals.at[pl.ds(lo * RPW, n)], sems[g % 2])

    gd = [None] * HNG
    gd[0] = fire(0)
    gd[1] = fire(1)

    def drain(g):
        gd[g].wait()
        lo, hi = HGROUPS[g], HGROUPS[g + 1]

        def acc_fn(r, carry):
            o = r * LANES
            acc = outv[pl.ds(o, LANES)]
            for f in range(lo, hi):
                acc = acc + vals[pl.ds(f * RPW + o, LANES)]
            outv[pl.ds(o, LANES)] = acc
            return carry

        lax.fori_loop(0, CHUNKS, acc_fn, 0)

    drain(0)
    gd[2] = fire(2)
    drain(1)
    drain(2)
    pltpu.sync_copy(outv, out_hbm.at[pl.ds(base, RPW)])


@functools.cache
def _build_sc_gather_half(f0):
    mesh = plsc.VectorSubcoreMesh(
        core_axis_name="c", subcore_axis_name="s",
        num_cores=NC, num_subcores=NSUB)
    return pl.kernel(
        functools.partial(_sc_gather_half_body, f0),
        out_type=jax.ShapeDtypeStruct((B,), jnp.float32),
        mesh=mesh,
        compiler_params=pltpu.CompilerParams(needs_layout_passes=False),
        scratch_types=[
            pltpu.VMEM((HALF * RPW,), jnp.int32),
            pltpu.VMEM((HALF * RPW,), jnp.float32),
            pltpu.VMEM((RPW,), jnp.float32),
            pltpu.SemaphoreType.DMA,
            pltpu.SemaphoreType.DMA,
        ],
    )


def kernel(raw_feat, sparse_tables, W_dense):
    rawT = raw_feat.T
    idx_all, densep = _build_tc_prep()(rawT, W_dense)
    ta = sparse_tables[:HALF].reshape(HALF, VOCAB)
    ta = lax.optimization_barrier(ta)
    table_a = ta.reshape(HALF * VOCAB)
    tb = sparse_tables[HALF:].reshape(HALF, VOCAB)
    tb = lax.optimization_barrier(tb)
    table_b = tb.reshape(HALF * VOCAB)
    part = _build_sc_gather_half(0)(idx_all, densep, table_a)
    return _build_sc_gather_half(HALF)(idx_all, part, table_b)
